# baseline (speedup 1.0000x reference)
import functools

import jax
import jax.numpy as jnp
from jax import lax
from jax.experimental import pallas as pl
from jax.experimental.pallas import tpu as pltpu
from jax.experimental.pallas import tpu_sc as plsc

_LAMBDA_S = 0.01
_K = 8
_N = 8192
_BR = 1024

_NC = 2
_NS = 16
_NW = _NC * _NS
_L = 16
_EPW = _N * _K // _NW


_NCHUNK = _N // 128


def _topk_body(x_ref, xt_ref, idx_ref, d2_ref):
    rp = x_ref[...]
    cp2 = xt_ref[...]
    dotn = lax.dot_general(rp, cp2, (((1,), (0,)), ((), ())),
                           preferred_element_type=jnp.float32)
    rsq = jnp.sum(rp * rp, axis=1, keepdims=True)
    csq = jnp.sum(cp2 * cp2, axis=0, keepdims=True) * 0.25
    d2 = jnp.maximum(rsq + csq + dotn, 0.0)
    kq = lax.bitcast_convert_type(d2, jnp.int32) & ~jnp.int32(8191)
    colid = lax.broadcasted_iota(jnp.int32, (1, _N), 1)
    imax = jnp.full((_BR, 128), jnp.iinfo(jnp.int32).max, jnp.int32)
    f = imax
    g = imax
    h = imax
    for c in range(0, _NCHUNK, 2):
        va = kq[:, c * 128:(c + 1) * 128] | colid[:, c * 128:(c + 1) * 128]
        vb = (kq[:, (c + 1) * 128:(c + 2) * 128]
              | colid[:, (c + 1) * 128:(c + 2) * 128])
        v = jnp.minimum(va, vb)
        f2 = jnp.minimum(f, v)
        w = jnp.maximum(f, v)
        g2 = jnp.minimum(g, w)
        w2 = jnp.maximum(g, w)
        h = jnp.minimum(h, w2)
        f = f2
        g = g2
    idx_cols = []
    d2_cols = []
    for k in range(_K + 1):
        m = jnp.min(f, axis=1, keepdims=True)
        if k > 0:
            idx_cols.append(m & jnp.int32(8191))
            d2_cols.append(lax.bitcast_convert_type(
                (m & ~jnp.int32(8191)) | jnp.int32(4096), jnp.float32))
        if k < _K:
            pick = f == m
            f = jnp.where(pick, g, f)
            g = jnp.where(pick, h, g)
            h = jnp.where(pick, jnp.iinfo(jnp.int32).max, h)
    idx_ref[...] = jnp.concatenate(idx_cols, axis=1)
    d2_ref[...] = jnp.concatenate(d2_cols, axis=1)


def _tc_topk(xpad, xpad_t):
    return pl.pallas_call(
        _topk_body,
        grid=(_N // _BR,),
        in_specs=[
            pl.BlockSpec((_BR, 8), lambda i: (i, 0)),
            pl.BlockSpec((8, _N), lambda i: (0, 0)),
        ],
        out_specs=[
            pl.BlockSpec((_BR, _K), lambda i: (i, 0)),
            pl.BlockSpec((_BR, _K), lambda i: (i, 0)),
        ],
        out_shape=[
            jax.ShapeDtypeStruct((_N, _K), jnp.int32),
            jax.ShapeDtypeStruct((_N, _K), jnp.float32),
        ],
        compiler_params=pltpu.CompilerParams(
            dimension_semantics=("parallel",)),
    )(xpad, xpad_t)


def _sc_body(w_hbm, s0_hbm, s1_hbm, s2_hbm, idx_hbm, d2_hbm, out_hbm,
             w_v, s0_v, s1_v, s2_v, idx_v, d2_v, acc_v):
    wid = lax.axis_index("s") * _NC + lax.axis_index("c")
    base = wid * _EPW
    pltpu.sync_copy(w_hbm, w_v)
    pltpu.sync_copy(s0_hbm, s0_v)
    pltpu.sync_copy(s1_hbm, s1_v)
    pltpu.sync_copy(s2_hbm, s2_v)
    pltpu.sync_copy(idx_hbm.at[pl.ds(base, _EPW)], idx_v)
    pltpu.sync_copy(d2_hbm.at[pl.ds(base, _EPW)], d2_v)

    lanes = lax.iota(jnp.int32, _L)

    def step(t, acc):
        off = t * _L
        jv = idx_v[pl.ds(off, _L)]
        civ = lax.shift_right_logical(base + off + lanes, 3)
        wj = plsc.load_gather(w_v, [jv])
        wi = plsc.load_gather(w_v, [civ])
        a0j = plsc.load_gather(s0_v, [jv])
        a0i = plsc.load_gather(s0_v, [civ])
        a1j = plsc.load_gather(s1_v, [jv])
        a1i = plsc.load_gather(s1_v, [civ])
        a2j = plsc.load_gather(s2_v, [jv])
        a2i = plsc.load_gather(s2_v, [civ])
        d2v = d2_v[pl.ds(off, _L)]
        denom = jnp.maximum(d2v, 1e-12)
        dw = wi - wj
        d0 = a0i - a0j
        d1 = a1i - a1j
        d2_ = a2i - a2j
        num = dw * dw + d0 * d0 + d1 * d1 + d2_ * d2_
        return acc + num / denom

    acc = lax.fori_loop(0, _EPW // _L, step, jnp.zeros((_L,), jnp.float32))
    acc_v[...] = acc
    pltpu.sync_copy(acc_v, out_hbm.at[wid])


def _sc_reduce(w, s0, s1, s2, idx_flat, d2_flat):
    mesh = plsc.VectorSubcoreMesh(core_axis_name="c", subcore_axis_name="s")
    fn = functools.partial(
        pl.kernel,
        mesh=mesh,
        compiler_params=pltpu.CompilerParams(needs_layout_passes=False),
        out_type=jax.ShapeDtypeStruct((_NW, _L), jnp.float32),
        scratch_types=[
            pltpu.VMEM((_N,), jnp.float32),
            pltpu.VMEM((_N,), jnp.float32),
            pltpu.VMEM((_N,), jnp.float32),
            pltpu.VMEM((_N,), jnp.float32),
            pltpu.VMEM((_EPW,), jnp.int32),
            pltpu.VMEM((_EPW,), jnp.float32),
            pltpu.VMEM((_L,), jnp.float32),
        ],
    )(_sc_body)
    return fn(w, s0, s1, s2, idx_flat, d2_flat)


def kernel(positions, weights, log_scales):
    xpad = jnp.zeros((_N, 8), jnp.float32).at[:, :3].set(positions)
    nbr_idx, nbr_d2 = _tc_topk(xpad, -2.0 * xpad.T)
    partials = _sc_reduce(
        weights,
        log_scales[:, 0],
        log_scales[:, 1],
        log_scales[:, 2],
        nbr_idx.reshape(-1),
        nbr_d2.reshape(-1),
    )
    num_edges = _N * _K
    return _LAMBDA_S * jnp.sum(partials) / num_edges


# speedup vs baseline: 1.1164x; 1.1164x over previous
import functools

import jax
import jax.numpy as jnp
from jax import lax
from jax.experimental import pallas as pl
from jax.experimental.pallas import tpu as pltpu
from jax.experimental.pallas import tpu_sc as plsc

_LAMBDA_S = 0.01
_K = 8
_N = 8192
_BR = 1024

_NC = 2
_NS = 16
_NW = _NC * _NS
_L = 16
_EPW = _N * _K // _NW


_NCHUNK = _N // 128


def _topk_body(x_ref, xt_ref, idx_ref, d2_ref):
    rp = x_ref[...]
    cp2 = xt_ref[...]
    dotn = lax.dot_general(rp, cp2, (((1,), (0,)), ((), ())),
                           preferred_element_type=jnp.float32)
    rsq = jnp.sum(rp * rp, axis=1, keepdims=True)
    csq = jnp.sum(cp2 * cp2, axis=0, keepdims=True) * 0.25
    d2 = jnp.maximum(rsq + csq + dotn, 0.0)
    kq = lax.bitcast_convert_type(d2, jnp.int32) & ~jnp.int32(8191)
    colid = lax.broadcasted_iota(jnp.int32, (1, _N), 1)
    imax = jnp.full((_BR, 128), jnp.iinfo(jnp.int32).max, jnp.int32)
    f = imax
    g = imax
    h = imax
    for c in range(0, _NCHUNK, 4):
        vs = [kq[:, i * 128:(i + 1) * 128] | colid[:, i * 128:(i + 1) * 128]
              for i in range(c, c + 4)]
        v = jnp.minimum(jnp.minimum(vs[0], vs[1]),
                        jnp.minimum(vs[2], vs[3]))
        f2 = jnp.minimum(f, v)
        w = jnp.maximum(f, v)
        g2 = jnp.minimum(g, w)
        w2 = jnp.maximum(g, w)
        h = jnp.minimum(h, w2)
        f = f2
        g = g2
    idx_cols = []
    d2_cols = []
    for k in range(_K + 1):
        m = jnp.min(f, axis=1, keepdims=True)
        if k > 0:
            idx_cols.append(m & jnp.int32(8191))
            d2_cols.append(lax.bitcast_convert_type(
                (m & ~jnp.int32(8191)) | jnp.int32(4096), jnp.float32))
        if k < _K:
            pick = f == m
            f = jnp.where(pick, g, f)
            g = jnp.where(pick, h, g)
            h = jnp.where(pick, jnp.iinfo(jnp.int32).max, h)
    idx_ref[...] = jnp.concatenate(idx_cols, axis=1)
    d2_ref[...] = jnp.concatenate(d2_cols, axis=1)


def _tc_topk(xpad, xpad_t):
    return pl.pallas_call(
        _topk_body,
        grid=(_N // _BR,),
        in_specs=[
            pl.BlockSpec((_BR, 8), lambda i: (i, 0)),
            pl.BlockSpec((8, _N), lambda i: (0, 0)),
        ],
        out_specs=[
            pl.BlockSpec((_BR, _K), lambda i: (i, 0)),
            pl.BlockSpec((_BR, _K), lambda i: (i, 0)),
        ],
        out_shape=[
            jax.ShapeDtypeStruct((_N, _K), jnp.int32),
            jax.ShapeDtypeStruct((_N, _K), jnp.float32),
        ],
        compiler_params=pltpu.CompilerParams(
            dimension_semantics=("parallel",)),
    )(xpad, xpad_t)


def _sc_body(w_hbm, s0_hbm, s1_hbm, s2_hbm, idx_hbm, d2_hbm, out_hbm,
             w_v, s0_v, s1_v, s2_v, idx_v, d2_v, acc_v):
    wid = lax.axis_index("s") * _NC + lax.axis_index("c")
    base = wid * _EPW
    pltpu.sync_copy(w_hbm, w_v)
    pltpu.sync_copy(s0_hbm, s0_v)
    pltpu.sync_copy(s1_hbm, s1_v)
    pltpu.sync_copy(s2_hbm, s2_v)
    pltpu.sync_copy(idx_hbm.at[pl.ds(base, _EPW)], idx_v)
    pltpu.sync_copy(d2_hbm.at[pl.ds(base, _EPW)], d2_v)

    lanes = lax.iota(jnp.int32, _L)

    def step(t, acc):
        off = t * _L
        jv = idx_v[pl.ds(off, _L)]
        civ = lax.shift_right_logical(base + off + lanes, 3)
        wj = plsc.load_gather(w_v, [jv])
        wi = plsc.load_gather(w_v, [civ])
        a0j = plsc.load_gather(s0_v, [jv])
        a0i = plsc.load_gather(s0_v, [civ])
        a1j = plsc.load_gather(s1_v, [jv])
        a1i = plsc.load_gather(s1_v, [civ])
        a2j = plsc.load_gather(s2_v, [jv])
        a2i = plsc.load_gather(s2_v, [civ])
        d2v = d2_v[pl.ds(off, _L)]
        denom = jnp.maximum(d2v, 1e-12)
        dw = wi - wj
        d0 = a0i - a0j
        d1 = a1i - a1j
        d2_ = a2i - a2j
        num = dw * dw + d0 * d0 + d1 * d1 + d2_ * d2_
        return acc + num / denom

    acc = lax.fori_loop(0, _EPW // _L, step, jnp.zeros((_L,), jnp.float32))
    acc_v[...] = acc
    pltpu.sync_copy(acc_v, out_hbm.at[wid])


def _sc_reduce(w, s0, s1, s2, idx_flat, d2_flat):
    mesh = plsc.VectorSubcoreMesh(core_axis_name="c", subcore_axis_name="s")
    fn = functools.partial(
        pl.kernel,
        mesh=mesh,
        compiler_params=pltpu.CompilerParams(needs_layout_passes=False),
        out_type=jax.ShapeDtypeStruct((_NW, _L), jnp.float32),
        scratch_types=[
            pltpu.VMEM((_N,), jnp.float32),
            pltpu.VMEM((_N,), jnp.float32),
            pltpu.VMEM((_N,), jnp.float32),
            pltpu.VMEM((_N,), jnp.float32),
            pltpu.VMEM((_EPW,), jnp.int32),
            pltpu.VMEM((_EPW,), jnp.float32),
            pltpu.VMEM((_L,), jnp.float32),
        ],
    )(_sc_body)
    return fn(w, s0, s1, s2, idx_flat, d2_flat)


def kernel(positions, weights, log_scales):
    xpad = jnp.zeros((_N, 8), jnp.float32).at[:, :3].set(positions)
    nbr_idx, nbr_d2 = _tc_topk(xpad, -2.0 * xpad.T)
    partials = _sc_reduce(
        weights,
        log_scales[:, 0],
        log_scales[:, 1],
        log_scales[:, 2],
        nbr_idx.reshape(-1),
        nbr_d2.reshape(-1),
    )
    num_edges = _N * _K
    return _LAMBDA_S * jnp.sum(partials) / num_edges


# speedup vs baseline: 1.3143x; 1.1773x over previous
import functools

import jax
import jax.numpy as jnp
from jax import lax
from jax.experimental import pallas as pl
from jax.experimental.pallas import tpu as pltpu
from jax.experimental.pallas import tpu_sc as plsc

_LAMBDA_S = 0.01
_K = 8
_N = 8192
_BR = 1024

_NC = 2
_NS = 16
_NW = _NC * _NS
_L = 16
_EPW = _N * _K // _NW


_NCHUNK = _N // 128


def _topk_body(x_ref, xt_ref, idx_ref, d2_ref):
    rp = x_ref[...]
    cp2 = xt_ref[...]
    dotn = lax.dot_general(rp, cp2, (((1,), (0,)), ((), ())),
                           preferred_element_type=jnp.float32)
    rsq = jnp.sum(rp * rp, axis=1, keepdims=True)
    csq = jnp.sum(cp2 * cp2, axis=0, keepdims=True) * 0.25
    d2 = jnp.maximum(rsq + csq + dotn, 0.0)
    kq = lax.bitcast_convert_type(d2, jnp.int32) & ~jnp.int32(8191)
    colb = lax.broadcasted_iota(jnp.int32, (1, _N), 1) + jnp.int32(1 << 23)
    inf = jnp.full((_BR, 128), jnp.inf, jnp.float32)
    f = inf
    g = inf
    h = inf
    for c in range(0, _NCHUNK, 4):
        vs = [lax.bitcast_convert_type(
                  kq[:, i * 128:(i + 1) * 128]
                  + colb[:, i * 128:(i + 1) * 128], jnp.float32)
              for i in range(c, c + 4)]
        v = jnp.minimum(jnp.minimum(vs[0], vs[1]),
                        jnp.minimum(vs[2], vs[3]))
        f2 = jnp.minimum(f, v)
        w = jnp.maximum(f, v)
        g2 = jnp.minimum(g, w)
        w2 = jnp.maximum(g, w)
        h = jnp.minimum(h, w2)
        f = f2
        g = g2
    idx_cols = []
    d2_cols = []
    for k in range(_K + 1):
        m = jnp.min(f, axis=1, keepdims=True)
        if k > 0:
            mi = lax.bitcast_convert_type(m, jnp.int32) - jnp.int32(1 << 23)
            idx_cols.append(mi & jnp.int32(8191))
            d2_cols.append(lax.bitcast_convert_type(
                (mi & ~jnp.int32(8191)) | jnp.int32(4096), jnp.float32))
        if k < _K:
            pick = f == m
            f = jnp.where(pick, g, f)
            g = jnp.where(pick, h, g)
            h = jnp.where(pick, jnp.float32(jnp.inf), h)
    idx_ref[...] = jnp.concatenate(idx_cols, axis=1)
    d2_ref[...] = jnp.concatenate(d2_cols, axis=1)


def _tc_topk(xpad, xpad_t):
    return pl.pallas_call(
        _topk_body,
        grid=(_N // _BR,),
        in_specs=[
            pl.BlockSpec((_BR, 8), lambda i: (i, 0)),
            pl.BlockSpec((8, _N), lambda i: (0, 0)),
        ],
        out_specs=[
            pl.BlockSpec((_BR, _K), lambda i: (i, 0)),
            pl.BlockSpec((_BR, _K), lambda i: (i, 0)),
        ],
        out_shape=[
            jax.ShapeDtypeStruct((_N, _K), jnp.int32),
            jax.ShapeDtypeStruct((_N, _K), jnp.float32),
        ],
        compiler_params=pltpu.CompilerParams(
            dimension_semantics=("parallel",)),
    )(xpad, xpad_t)


def _sc_body(w_hbm, s0_hbm, s1_hbm, s2_hbm, idx_hbm, d2_hbm, out_hbm,
             w_v, s0_v, s1_v, s2_v, idx_v, d2_v, acc_v):
    wid = lax.axis_index("s") * _NC + lax.axis_index("c")
    base = wid * _EPW
    pltpu.sync_copy(w_hbm, w_v)
    pltpu.sync_copy(s0_hbm, s0_v)
    pltpu.sync_copy(s1_hbm, s1_v)
    pltpu.sync_copy(s2_hbm, s2_v)
    pltpu.sync_copy(idx_hbm.at[pl.ds(base, _EPW)], idx_v)
    pltpu.sync_copy(d2_hbm.at[pl.ds(base, _EPW)], d2_v)

    lanes = lax.iota(jnp.int32, _L)

    def step(t, acc):
        off = t * _L
        jv = idx_v[pl.ds(off, _L)]
        civ = lax.shift_right_logical(base + off + lanes, 3)
        wj = plsc.load_gather(w_v, [jv])
        wi = plsc.load_gather(w_v, [civ])
        a0j = plsc.load_gather(s0_v, [jv])
        a0i = plsc.load_gather(s0_v, [civ])
        a1j = plsc.load_gather(s1_v, [jv])
        a1i = plsc.load_gather(s1_v, [civ])
        a2j = plsc.load_gather(s2_v, [jv])
        a2i = plsc.load_gather(s2_v, [civ])
        d2v = d2_v[pl.ds(off, _L)]
        denom = jnp.maximum(d2v, 1e-12)
        dw = wi - wj
        d0 = a0i - a0j
        d1 = a1i - a1j
        d2_ = a2i - a2j
        num = dw * dw + d0 * d0 + d1 * d1 + d2_ * d2_
        return acc + num / denom

    acc = lax.fori_loop(0, _EPW // _L, step, jnp.zeros((_L,), jnp.float32))
    acc_v[...] = acc
    pltpu.sync_copy(acc_v, out_hbm.at[wid])


def _sc_reduce(w, s0, s1, s2, idx_flat, d2_flat):
    mesh = plsc.VectorSubcoreMesh(core_axis_name="c", subcore_axis_name="s")
    fn = functools.partial(
        pl.kernel,
        mesh=mesh,
        compiler_params=pltpu.CompilerParams(needs_layout_passes=False),
        out_type=jax.ShapeDtypeStruct((_NW, _L), jnp.float32),
        scratch_types=[
            pltpu.VMEM((_N,), jnp.float32),
            pltpu.VMEM((_N,), jnp.float32),
            pltpu.VMEM((_N,), jnp.float32),
            pltpu.VMEM((_N,), jnp.float32),
            pltpu.VMEM((_EPW,), jnp.int32),
            pltpu.VMEM((_EPW,), jnp.float32),
            pltpu.VMEM((_L,), jnp.float32),
        ],
    )(_sc_body)
    return fn(w, s0, s1, s2, idx_flat, d2_flat)


def kernel(positions, weights, log_scales):
    xpad = jnp.zeros((_N, 8), jnp.float32).at[:, :3].set(positions)
    nbr_idx, nbr_d2 = _tc_topk(xpad, -2.0 * xpad.T)
    partials = _sc_reduce(
        weights,
        log_scales[:, 0],
        log_scales[:, 1],
        log_scales[:, 2],
        nbr_idx.reshape(-1),
        nbr_d2.reshape(-1),
    )
    num_edges = _N * _K
    return _LAMBDA_S * jnp.sum(partials) / num_edges
